# baseline (speedup 1.0000x reference)
import functools

import jax
import jax.numpy as jnp
from jax import lax
from jax.experimental import pallas as pl
from jax.experimental.pallas import tpu as pltpu
from jax.experimental.pallas import tpu_sc as plsc

N = 10000
E = 320000
H = 128
NUM_LAYERS = 2

NC = 2
NS = 16
NW = NC * NS
CH = 64
NCHUNK = -(-(E // NW) // CH)
EDGES_PER_TILE = CH * NCHUNK
E_PAD = EDGES_PER_TILE * NW
NPAD = 10240
ROWS_PER_TILE = NPAD // NS


def _sc_segment_sum_body(x_hbm, src_hbm, dst_hbm, zero_hbm, out_hbm,
                         si0, si1, di0, di1, rows0, rows1, acc_sh,
                         sg0, sg1):
  core = lax.axis_index("c")
  sub = lax.axis_index("s")
  wid = sub * NC + core
  base = wid * EDGES_PER_TILE

  pltpu.sync_copy(zero_hbm, acc_sh.at[pl.ds(sub * ROWS_PER_TILE, ROWS_PER_TILE)])
  plsc.subcore_barrier()

  pltpu.sync_copy(src_hbm.at[pl.ds(base, CH)], si0)
  pltpu.async_copy(x_hbm.at[si0], rows0, sg0)
  pltpu.sync_copy(src_hbm.at[pl.ds(base + CH, CH)], si1)
  pltpu.async_copy(x_hbm.at[si1], rows1, sg1)

  def chunk_step(c, si, di, rows, sg, start_next):
    off = base + c * CH
    pltpu.sync_copy(dst_hbm.at[pl.ds(off, CH)], di)
    pltpu.make_async_copy(x_hbm.at[si], rows, sg).wait()
    pltpu.sync_copy(rows, acc_sh.at[di], add=True)
    if start_next:
      pltpu.sync_copy(src_hbm.at[pl.ds(off + 2 * CH, CH)], si)
      pltpu.async_copy(x_hbm.at[si], rows, sg)

  def body(g, carry):
    chunk_step(2 * g, si0, di0, rows0, sg0, True)
    chunk_step(2 * g + 1, si1, di1, rows1, sg1, True)
    return carry

  lax.fori_loop(0, NCHUNK // 2 - 1, body, 0)
  if NCHUNK % 2:
    chunk_step(NCHUNK - 3, si0, di0, rows0, sg0, True)
    chunk_step(NCHUNK - 2, si1, di1, rows1, sg1, False)
    chunk_step(NCHUNK - 1, si0, di0, rows0, sg0, False)
  else:
    chunk_step(NCHUNK - 2, si0, di0, rows0, sg0, False)
    chunk_step(NCHUNK - 1, si1, di1, rows1, sg1, False)
  plsc.subcore_barrier()

  r0 = sub * ROWS_PER_TILE
  pltpu.sync_copy(acc_sh.at[pl.ds(r0, ROWS_PER_TILE)],
                  out_hbm.at[core, pl.ds(r0, ROWS_PER_TILE)])


_sc_segment_sum = functools.partial(
    pl.kernel,
    mesh=plsc.VectorSubcoreMesh(core_axis_name="c", subcore_axis_name="s"),
    out_type=jax.ShapeDtypeStruct((NC, NPAD, H), jnp.float32),
    scratch_types=[
        pltpu.VMEM((CH,), jnp.int32),
        pltpu.VMEM((CH,), jnp.int32),
        pltpu.VMEM((CH,), jnp.int32),
        pltpu.VMEM((CH,), jnp.int32),
        pltpu.VMEM((CH, H), jnp.float32),
        pltpu.VMEM((CH, H), jnp.float32),
        pltpu.VMEM_SHARED((NPAD, H), jnp.float32),
        pltpu.SemaphoreType.DMA,
        pltpu.SemaphoreType.DMA,
    ],
)(_sc_segment_sum_body)


def _tc_dense_body(x_ref, part_ref, w0_ref, w1_ref, pp_ref, out_ref):
  def bn_relu(m, g_row, b_row):
    mean = jnp.mean(m, axis=0, keepdims=True)
    c = m - mean
    var = jnp.mean(c * c, axis=0, keepdims=True)
    y = c * lax.rsqrt(var + 1e-5) * pp_ref[g_row:g_row + 1, :] \
        + pp_ref[b_row:b_row + 1, :]
    return jnp.maximum(y, 0.0)

  neigh = part_ref[0, 0:N, :] + part_ref[1, 0:N, :]
  r = (1.0 + pp_ref[6:7, :]) * x_ref[...] + neigh
  m = jnp.dot(r, w0_ref[...], preferred_element_type=jnp.float32)
  m = bn_relu(m, 0, 1)
  m = jnp.dot(m, w1_ref[...], preferred_element_type=jnp.float32)
  m = bn_relu(m, 2, 3)
  out_ref[...] = bn_relu(m, 4, 5)


_tc_dense = pl.pallas_call(
    _tc_dense_body,
    out_shape=jax.ShapeDtypeStruct((N, H), jnp.float32),
)


def kernel(h, edge_index, params):
  pad = E_PAD - E
  src = jnp.concatenate([edge_index[0], jnp.zeros((pad,), jnp.int32)])
  dst = jnp.concatenate(
      [edge_index[1], N + (jnp.arange(pad, dtype=jnp.int32) % (NPAD - N))])
  zero = jnp.zeros((ROWS_PER_TILE, H), jnp.float32)

  outs = [h]
  x = h
  for i in range(NUM_LAYERS):
    partials = _sc_segment_sum(x, src, dst, zero)
    pp = jnp.stack([
        params[f"g_mlp_{i}"], params[f"b_mlp_{i}"],
        params[f"g_app_{i}"], params[f"b_app_{i}"],
        params[f"g_enc_{i}"], params[f"b_enc_{i}"],
        jnp.full((H,), params[f"eps_{i}"], jnp.float32),
        jnp.zeros((H,), jnp.float32),
    ])
    x = _tc_dense(x, partials, params[f"W0_{i}"], params[f"W1_{i}"], pp)
    outs.append(x)
  return jnp.concatenate([t.reshape(1, N, H) for t in outs], axis=-1)


# speedup vs baseline: 1.2996x; 1.2996x over previous
import functools

import jax
import jax.numpy as jnp
from jax import lax
from jax.experimental import pallas as pl
from jax.experimental.pallas import tpu as pltpu
from jax.experimental.pallas import tpu_sc as plsc

N = 10000
E = 320000
H = 128
NUM_LAYERS = 2

NC = 2
NS = 16
NW = NC * NS
CH = 80
NCHUNK = -(-(E // NW) // CH)
EDGES_PER_TILE = CH * NCHUNK
E_PAD = EDGES_PER_TILE * NW
NPAD = 10240
ROWS_PER_TILE = NPAD // NS


def _sc_segment_sum_body(x_hbm, src_hbm, dst_hbm, zero_hbm, out_hbm,
                         si0, si1, di0, di1, rows0, rows1, acc_sh,
                         sg0, sg1):
  core = lax.axis_index("c")
  sub = lax.axis_index("s")
  wid = sub * NC + core
  base = wid * EDGES_PER_TILE

  pltpu.sync_copy(zero_hbm, acc_sh.at[pl.ds(sub * ROWS_PER_TILE, ROWS_PER_TILE)])
  plsc.subcore_barrier()

  pltpu.sync_copy(src_hbm.at[pl.ds(base, CH)], si0)
  pltpu.async_copy(x_hbm.at[si0], rows0, sg0)
  pltpu.sync_copy(src_hbm.at[pl.ds(base + CH, CH)], si1)
  pltpu.async_copy(x_hbm.at[si1], rows1, sg1)

  def chunk_step(c, si, di, rows, sg, start_next):
    off = base + c * CH
    pltpu.sync_copy(dst_hbm.at[pl.ds(off, CH)], di)
    pltpu.make_async_copy(x_hbm.at[si], rows, sg).wait()
    pltpu.sync_copy(rows, acc_sh.at[di], add=True)
    if start_next:
      pltpu.sync_copy(src_hbm.at[pl.ds(off + 2 * CH, CH)], si)
      pltpu.async_copy(x_hbm.at[si], rows, sg)

  def body(g, carry):
    chunk_step(2 * g, si0, di0, rows0, sg0, True)
    chunk_step(2 * g + 1, si1, di1, rows1, sg1, True)
    return carry

  lax.fori_loop(0, NCHUNK // 2 - 1, body, 0)
  if NCHUNK % 2:
    chunk_step(NCHUNK - 3, si0, di0, rows0, sg0, True)
    chunk_step(NCHUNK - 2, si1, di1, rows1, sg1, False)
    chunk_step(NCHUNK - 1, si0, di0, rows0, sg0, False)
  else:
    chunk_step(NCHUNK - 2, si0, di0, rows0, sg0, False)
    chunk_step(NCHUNK - 1, si1, di1, rows1, sg1, False)
  plsc.subcore_barrier()

  r0 = sub * ROWS_PER_TILE
  pltpu.sync_copy(acc_sh.at[pl.ds(r0, ROWS_PER_TILE)],
                  out_hbm.at[core, pl.ds(r0, ROWS_PER_TILE)])


_sc_segment_sum = functools.partial(
    pl.kernel,
    mesh=plsc.VectorSubcoreMesh(core_axis_name="c", subcore_axis_name="s"),
    out_type=jax.ShapeDtypeStruct((NC, NPAD, H), jnp.float32),
    scratch_types=[
        pltpu.VMEM((CH,), jnp.int32),
        pltpu.VMEM((CH,), jnp.int32),
        pltpu.VMEM((CH,), jnp.int32),
        pltpu.VMEM((CH,), jnp.int32),
        pltpu.VMEM((CH, H), jnp.float32),
        pltpu.VMEM((CH, H), jnp.float32),
        pltpu.VMEM_SHARED((NPAD, H), jnp.float32),
        pltpu.SemaphoreType.DMA,
        pltpu.SemaphoreType.DMA,
    ],
)(_sc_segment_sum_body)


def _tc_dense_body(x_ref, part_ref, w0_ref, w1_ref, pp_ref, out_ref):
  def bn_relu(m, g_row, b_row):
    mean = jnp.mean(m, axis=0, keepdims=True)
    c = m - mean
    var = jnp.mean(c * c, axis=0, keepdims=True)
    y = c * lax.rsqrt(var + 1e-5) * pp_ref[g_row:g_row + 1, :] \
        + pp_ref[b_row:b_row + 1, :]
    return jnp.maximum(y, 0.0)

  neigh = part_ref[0, 0:N, :] + part_ref[1, 0:N, :]
  r = (1.0 + pp_ref[6:7, :]) * x_ref[...] + neigh
  m = jnp.dot(r, w0_ref[...], preferred_element_type=jnp.float32)
  m = bn_relu(m, 0, 1)
  m = jnp.dot(m, w1_ref[...], preferred_element_type=jnp.float32)
  m = bn_relu(m, 2, 3)
  out_ref[...] = bn_relu(m, 4, 5)


_tc_dense = pl.pallas_call(
    _tc_dense_body,
    out_shape=jax.ShapeDtypeStruct((N, H), jnp.float32),
)


def kernel(h, edge_index, params):
  pad = E_PAD - E
  src = jnp.concatenate([edge_index[0], jnp.zeros((pad,), jnp.int32)])
  dst = jnp.concatenate(
      [edge_index[1], N + (jnp.arange(pad, dtype=jnp.int32) % (NPAD - N))])
  zero = jnp.zeros((ROWS_PER_TILE, H), jnp.float32)

  outs = [h]
  x = h
  for i in range(NUM_LAYERS):
    partials = _sc_segment_sum(x, src, dst, zero)
    pp = jnp.stack([
        params[f"g_mlp_{i}"], params[f"b_mlp_{i}"],
        params[f"g_app_{i}"], params[f"b_app_{i}"],
        params[f"g_enc_{i}"], params[f"b_enc_{i}"],
        jnp.full((H,), params[f"eps_{i}"], jnp.float32),
        jnp.zeros((H,), jnp.float32),
    ])
    x = _tc_dense(x, partials, params[f"W0_{i}"], params[f"W1_{i}"], pp)
    outs.append(x)
  return jnp.concatenate([t.reshape(1, N, H) for t in outs], axis=-1)


# speedup vs baseline: 1.5498x; 1.1925x over previous
import functools

import jax
import jax.numpy as jnp
from jax import lax
from jax.experimental import pallas as pl
from jax.experimental.pallas import tpu as pltpu
from jax.experimental.pallas import tpu_sc as plsc

N = 10000
E = 320000
H = 128
NUM_LAYERS = 2

NC = 2
NS = 16
NW = NC * NS
CH = 128
NCHUNK = -(-(E // NW) // CH)
EDGES_PER_TILE = CH * NCHUNK
E_PAD = EDGES_PER_TILE * NW
NPAD = 10240
ROWS_PER_TILE = NPAD // NS


def _sc_segment_sum_body(x_hbm, src_hbm, dst_hbm, zero_hbm, out_hbm,
                         si0, si1, di0, di1, rows0, rows1, acc_sh,
                         sg0, sg1):
  core = lax.axis_index("c")
  sub = lax.axis_index("s")
  wid = sub * NC + core
  base = wid * EDGES_PER_TILE

  pltpu.sync_copy(zero_hbm, acc_sh.at[pl.ds(sub * ROWS_PER_TILE, ROWS_PER_TILE)])
  plsc.subcore_barrier()

  pltpu.sync_copy(src_hbm.at[pl.ds(base, CH)], si0)
  pltpu.async_copy(x_hbm.at[si0], rows0, sg0)
  pltpu.sync_copy(src_hbm.at[pl.ds(base + CH, CH)], si1)
  pltpu.async_copy(x_hbm.at[si1], rows1, sg1)

  def chunk_step(c, si, di, rows, sg, start_next):
    off = base + c * CH
    pltpu.sync_copy(dst_hbm.at[pl.ds(off, CH)], di)
    pltpu.make_async_copy(x_hbm.at[si], rows, sg).wait()
    pltpu.sync_copy(rows, acc_sh.at[di], add=True)
    if start_next:
      pltpu.sync_copy(src_hbm.at[pl.ds(off + 2 * CH, CH)], si)
      pltpu.async_copy(x_hbm.at[si], rows, sg)

  def body(g, carry):
    chunk_step(2 * g, si0, di0, rows0, sg0, True)
    chunk_step(2 * g + 1, si1, di1, rows1, sg1, True)
    return carry

  lax.fori_loop(0, NCHUNK // 2 - 1, body, 0)
  if NCHUNK % 2:
    chunk_step(NCHUNK - 3, si0, di0, rows0, sg0, True)
    chunk_step(NCHUNK - 2, si1, di1, rows1, sg1, False)
    chunk_step(NCHUNK - 1, si0, di0, rows0, sg0, False)
  else:
    chunk_step(NCHUNK - 2, si0, di0, rows0, sg0, False)
    chunk_step(NCHUNK - 1, si1, di1, rows1, sg1, False)
  plsc.subcore_barrier()

  r0 = sub * ROWS_PER_TILE
  pltpu.sync_copy(acc_sh.at[pl.ds(r0, ROWS_PER_TILE)],
                  out_hbm.at[core, pl.ds(r0, ROWS_PER_TILE)])


_sc_segment_sum = functools.partial(
    pl.kernel,
    mesh=plsc.VectorSubcoreMesh(core_axis_name="c", subcore_axis_name="s"),
    out_type=jax.ShapeDtypeStruct((NC, NPAD, H), jnp.float32),
    scratch_types=[
        pltpu.VMEM((CH,), jnp.int32),
        pltpu.VMEM((CH,), jnp.int32),
        pltpu.VMEM((CH,), jnp.int32),
        pltpu.VMEM((CH,), jnp.int32),
        pltpu.VMEM((CH, H), jnp.float32),
        pltpu.VMEM((CH, H), jnp.float32),
        pltpu.VMEM_SHARED((NPAD, H), jnp.float32),
        pltpu.SemaphoreType.DMA,
        pltpu.SemaphoreType.DMA,
    ],
)(_sc_segment_sum_body)


def _tc_dense_body(x_ref, part_ref, w0_ref, w1_ref, pp_ref, out_ref):
  def bn_relu(m, g_row, b_row):
    mean = jnp.mean(m, axis=0, keepdims=True)
    c = m - mean
    var = jnp.mean(c * c, axis=0, keepdims=True)
    y = c * lax.rsqrt(var + 1e-5) * pp_ref[g_row:g_row + 1, :] \
        + pp_ref[b_row:b_row + 1, :]
    return jnp.maximum(y, 0.0)

  neigh = part_ref[0, 0:N, :] + part_ref[1, 0:N, :]
  r = (1.0 + pp_ref[6:7, :]) * x_ref[...] + neigh
  m = jnp.dot(r, w0_ref[...], preferred_element_type=jnp.float32)
  m = bn_relu(m, 0, 1)
  m = jnp.dot(m, w1_ref[...], preferred_element_type=jnp.float32)
  m = bn_relu(m, 2, 3)
  out_ref[...] = bn_relu(m, 4, 5)


_tc_dense = pl.pallas_call(
    _tc_dense_body,
    out_shape=jax.ShapeDtypeStruct((N, H), jnp.float32),
)


def kernel(h, edge_index, params):
  pad = E_PAD - E
  src = jnp.concatenate(
      [edge_index[0], jnp.arange(pad, dtype=jnp.int32) % N])
  dst = jnp.concatenate(
      [edge_index[1], N + (jnp.arange(pad, dtype=jnp.int32) % (NPAD - N))])
  zero = jnp.zeros((ROWS_PER_TILE, H), jnp.float32)

  outs = [h]
  x = h
  for i in range(NUM_LAYERS):
    partials = _sc_segment_sum(x, src, dst, zero)
    pp = jnp.stack([
        params[f"g_mlp_{i}"], params[f"b_mlp_{i}"],
        params[f"g_app_{i}"], params[f"b_app_{i}"],
        params[f"g_enc_{i}"], params[f"b_enc_{i}"],
        jnp.full((H,), params[f"eps_{i}"], jnp.float32),
        jnp.zeros((H,), jnp.float32),
    ])
    x = _tc_dense(x, partials, params[f"W0_{i}"], params[f"W1_{i}"], pp)
    outs.append(x)
  return jnp.concatenate([t.reshape(1, N, H) for t in outs], axis=-1)


# speedup vs baseline: 1.7251x; 1.1131x over previous
import functools

import jax
import jax.numpy as jnp
from jax import lax
from jax.experimental import pallas as pl
from jax.experimental.pallas import tpu as pltpu
from jax.experimental.pallas import tpu_sc as plsc

N = 10000
E = 320000
H = 128
NUM_LAYERS = 2

NC = 2
NS = 16
NW = NC * NS
CH = 128
NCHUNK = -(-(E // NW) // CH)
EDGES_PER_TILE = CH * NCHUNK
E_PAD = EDGES_PER_TILE * NW
NPAD = 10240
ROWS_PER_TILE = NPAD // NS


def _sc_segment_sum_body(x_hbm, src_hbm, dst_hbm, zero_hbm, out_hbm,
                         si0, si1, di0, di1, rows0, rows1, acc_sh,
                         sg0, sg1, ss0, ss1, ssi0, ssi1, sdi0, sdi1):
  core = lax.axis_index("c")
  sub = lax.axis_index("s")
  wid = sub * NC + core
  base = wid * EDGES_PER_TILE

  si = (si0, si1)
  di = (di0, di1)
  rows = (rows0, rows1)
  sg = (sg0, sg1)
  ss = (ss0, ss1)
  ssi = (ssi0, ssi1)
  sdi = (sdi0, sdi1)

  pltpu.sync_copy(zero_hbm, acc_sh.at[pl.ds(sub * ROWS_PER_TILE, ROWS_PER_TILE)])
  plsc.subcore_barrier()

  def chunk_step(c, b, load_si2, do_next, wait_prev=True):
    o = 1 - b
    pltpu.make_async_copy(x_hbm.at[si[b]], rows[b], sg[b]).wait()
    if load_si2:
      pltpu.async_copy(src_hbm.at[pl.ds(base + (c + 2) * CH, CH)], si[b], ssi[b])
    pltpu.make_async_copy(dst_hbm.at[pl.ds(base + c * CH, CH)], di[b],
                          sdi[b]).wait()
    pltpu.async_copy(rows[b], acc_sh.at[di[b]], sem=ss[b], add=True)
    if wait_prev:
      pltpu.make_async_copy(rows[o], acc_sh.at[di[o]], ss[o]).wait()
    if do_next:
      pltpu.async_copy(dst_hbm.at[pl.ds(base + (c + 1) * CH, CH)], di[o], sdi[o])
      pltpu.make_async_copy(src_hbm.at[pl.ds(base + (c + 1) * CH, CH)],
                            si[o], ssi[o]).wait()
      pltpu.async_copy(x_hbm.at[si[o]], rows[o], sg[o])

  pltpu.sync_copy(src_hbm.at[pl.ds(base, CH)], si0)
  pltpu.async_copy(dst_hbm.at[pl.ds(base, CH)], di0, sdi0)
  pltpu.async_copy(x_hbm.at[si0], rows0, sg0)
  pltpu.async_copy(src_hbm.at[pl.ds(base + CH, CH)], si1, ssi1)

  chunk_step(0, 0, True, True, wait_prev=False)
  chunk_step(1, 1, True, True)

  def body(g, carry):
    chunk_step(2 * g, 0, True, True)
    chunk_step(2 * g + 1, 1, True, True)
    return carry

  lax.fori_loop(1, NCHUNK // 2 - 1, body, 0)
  chunk_step(NCHUNK - 2, 0, False, True)
  chunk_step(NCHUNK - 1, 1, False, False)
  pltpu.make_async_copy(rows1, acc_sh.at[di1], ss1).wait()
  plsc.subcore_barrier()

  r0 = sub * ROWS_PER_TILE
  pltpu.sync_copy(acc_sh.at[pl.ds(r0, ROWS_PER_TILE)],
                  out_hbm.at[core, pl.ds(r0, ROWS_PER_TILE)])


_sc_segment_sum = functools.partial(
    pl.kernel,
    mesh=plsc.VectorSubcoreMesh(core_axis_name="c", subcore_axis_name="s"),
    out_type=jax.ShapeDtypeStruct((NC, NPAD, H), jnp.float32),
    scratch_types=[
        pltpu.VMEM((CH,), jnp.int32),
        pltpu.VMEM((CH,), jnp.int32),
        pltpu.VMEM((CH,), jnp.int32),
        pltpu.VMEM((CH,), jnp.int32),
        pltpu.VMEM((CH, H), jnp.float32),
        pltpu.VMEM((CH, H), jnp.float32),
        pltpu.VMEM_SHARED((NPAD, H), jnp.float32),
        pltpu.SemaphoreType.DMA,
        pltpu.SemaphoreType.DMA,
        pltpu.SemaphoreType.DMA,
        pltpu.SemaphoreType.DMA,
        pltpu.SemaphoreType.DMA,
        pltpu.SemaphoreType.DMA,
        pltpu.SemaphoreType.DMA,
        pltpu.SemaphoreType.DMA,
    ],
)(_sc_segment_sum_body)


def _tc_dense_body(x_ref, part_ref, w0_ref, w1_ref, pp_ref, out_ref):
  def bn_relu(m, g_row, b_row):
    mean = jnp.mean(m, axis=0, keepdims=True)
    c = m - mean
    var = jnp.mean(c * c, axis=0, keepdims=True)
    y = c * lax.rsqrt(var + 1e-5) * pp_ref[g_row:g_row + 1, :] \
        + pp_ref[b_row:b_row + 1, :]
    return jnp.maximum(y, 0.0)

  neigh = part_ref[0, 0:N, :] + part_ref[1, 0:N, :]
  r = (1.0 + pp_ref[6:7, :]) * x_ref[...] + neigh
  m = jnp.dot(r, w0_ref[...], preferred_element_type=jnp.float32)
  m = bn_relu(m, 0, 1)
  m = jnp.dot(m, w1_ref[...], preferred_element_type=jnp.float32)
  m = bn_relu(m, 2, 3)
  out_ref[...] = bn_relu(m, 4, 5)


_tc_dense = pl.pallas_call(
    _tc_dense_body,
    out_shape=jax.ShapeDtypeStruct((N, H), jnp.float32),
)


def kernel(h, edge_index, params):
  pad = E_PAD - E
  src = jnp.concatenate(
      [edge_index[0], jnp.arange(pad, dtype=jnp.int32) % N])
  dst = jnp.concatenate(
      [edge_index[1], N + (jnp.arange(pad, dtype=jnp.int32) % (NPAD - N))])
  zero = jnp.zeros((ROWS_PER_TILE, H), jnp.float32)

  outs = [h]
  x = h
  for i in range(NUM_LAYERS):
    partials = _sc_segment_sum(x, src, dst, zero)
    pp = jnp.stack([
        params[f"g_mlp_{i}"], params[f"b_mlp_{i}"],
        params[f"g_app_{i}"], params[f"b_app_{i}"],
        params[f"g_enc_{i}"], params[f"b_enc_{i}"],
        jnp.full((H,), params[f"eps_{i}"], jnp.float32),
        jnp.zeros((H,), jnp.float32),
    ])
    x = _tc_dense(x, partials, params[f"W0_{i}"], params[f"W1_{i}"], pp)
    outs.append(x)
  return jnp.concatenate([t.reshape(1, N, H) for t in outs], axis=-1)
